# baseline (speedup 1.0000x reference)
import functools

import jax
import jax.numpy as jnp
from jax import lax
from jax.experimental import pallas as pl
from jax.experimental.pallas import tpu as pltpu
from jax.experimental.pallas import tpu_sc as plsc

VOCAB = 100000
MAXLEN = 2048
DIM = 768
B = 4
S = 2048
N = B * S

NSPLIT = 2
NP = N // NSPLIT
BH = B // NSPLIT

NC = 2
NS = 16
NW = NC * NS
ROWS_PER_W = NP // NW
CHUNK = 64
NCHUNK = ROWS_PER_W // CHUNK


def _sc_gather_kernel(piece, table_hbm, idx_hbm, out_hbm,
                      idx0, idx1, buf0, buf1, sem0, sem1):
  wid = lax.axis_index("s") * NC + lax.axis_index("c")
  base = pl.multiple_of(wid * ROWS_PER_W, ROWS_PER_W)

  idxs = (idx0, idx1)
  bufs = (buf0, buf1)
  sems = (sem0, sem1)

  def start(ci):
    off = pl.multiple_of(base + ci * CHUNK, CHUNK)
    slot = ci % 2
    grow = piece * NP + off
    b = lax.div(grow, S)
    col = pl.multiple_of(lax.rem(grow, S), CHUNK)
    pltpu.sync_copy(idx_hbm.at[b, pl.ds(col, CHUNK)], idxs[slot])
    return pltpu.async_copy(table_hbm.at[idxs[slot]], bufs[slot], sems[slot])

  cp = start(0)
  for ci in range(NCHUNK):
    nxt = start(ci + 1) if ci + 1 < NCHUNK else None
    cp.wait()
    off = pl.multiple_of(base + ci * CHUNK, CHUNK)
    pltpu.sync_copy(bufs[ci % 2], out_hbm.at[pl.ds(off, CHUNK)])
    cp = nxt


def _make_sc_gather(piece):
  mesh = plsc.VectorSubcoreMesh(core_axis_name="c", subcore_axis_name="s")
  k = functools.partial(
      pl.kernel, mesh=mesh,
      out_type=jax.ShapeDtypeStruct((NP, DIM), jnp.float32),
      scratch_types=[
          pltpu.VMEM((CHUNK,), jnp.int32),
          pltpu.VMEM((CHUNK,), jnp.int32),
          pltpu.VMEM((CHUNK, DIM), jnp.float32),
          pltpu.VMEM((CHUNK, DIM), jnp.float32),
          pltpu.SemaphoreType.DMA,
          pltpu.SemaphoreType.DMA,
      ],
  )(functools.partial(_sc_gather_kernel, piece))
  return k


BS = 256
SB = S // BS
PB = NP // BS


def _tc_ln_kernel(g_ref, pos_ref, tt_ref, w01_ref, gamma_ref, beta_ref,
                  out_ref):
  row0 = w01_ref[0, :]
  drow = w01_ref[1, :] - row0
  x = g_ref[...] + pos_ref[...]
  x = x + row0[None, :] + tt_ref[...] * drow[None, :]
  mean = jnp.mean(x, axis=-1, keepdims=True)
  xc = x - mean
  var = jnp.mean(xc * xc, axis=-1, keepdims=True)
  y = xc * lax.rsqrt(var + 1e-5)
  out_ref[...] = y * gamma_ref[...] + beta_ref[...]


def _tc_ln_kernel_aliased(g_ref, pos_ref, tt_ref, w01_ref, gamma_ref,
                          beta_ref, prev_ref, out_ref):
  del prev_ref
  _tc_ln_kernel(g_ref, pos_ref, tt_ref, w01_ref, gamma_ref, beta_ref,
                out_ref)


def _make_tc_ln(piece):
  first = piece == 0
  body = _tc_ln_kernel if first else _tc_ln_kernel_aliased
  in_specs = [
      pl.BlockSpec((BS, DIM), lambda s, b: (b * SB + s, 0)),
      pl.BlockSpec((BS, DIM), lambda s, b: (s, 0)),
      pl.BlockSpec((BS, 1),
                   lambda s, b, p=piece: ((p * BH + b) * SB + s, 0)),
      pl.BlockSpec((8, DIM), lambda s, b: (0, 0)),
      pl.BlockSpec((1, DIM), lambda s, b: (0, 0)),
      pl.BlockSpec((1, DIM), lambda s, b: (0, 0)),
  ]
  if not first:
    in_specs.append(pl.BlockSpec((8, DIM), lambda s, b: (0, 0)))
  return pl.pallas_call(
      body,
      grid=(SB, BH),
      in_specs=in_specs,
      out_specs=pl.BlockSpec(
          (BS, DIM), lambda s, b, p=piece: ((p * BH + b) * SB + s, 0)),
      out_shape=jax.ShapeDtypeStruct((N, DIM), jnp.float32),
      input_output_aliases={} if first else {6: 0},
  )


@jax.jit
def _run(input_ids, ttf, W_word, W_tt, gamma2d, beta2d):
  gs = [_make_sc_gather(p)(W_word, input_ids) for p in range(NSPLIT)]
  out = _make_tc_ln(0)(gs[0], W_tt, ttf, W_tt, gamma2d, beta2d)
  for p in range(1, NSPLIT):
    out = _make_tc_ln(p)(gs[p], W_tt, ttf, W_tt, gamma2d, beta2d, out)
  return out


def kernel(input_ids, token_type_ids, W_word, W_tt, gamma, beta):
  ids2d = input_ids.astype(jnp.int32)
  ttf = token_type_ids.reshape(N, 1).astype(jnp.float32)
  out = _run(ids2d, ttf, W_word, W_tt,
             gamma.reshape(1, DIM), beta.reshape(1, DIM))
  return out.reshape(B, S, DIM)


# speedup vs baseline: 1.1646x; 1.1646x over previous
import functools

import jax
import jax.numpy as jnp
from jax import lax
from jax.experimental import pallas as pl
from jax.experimental.pallas import tpu as pltpu
from jax.experimental.pallas import tpu_sc as plsc

VOCAB = 100000
MAXLEN = 2048
DIM = 768
B = 4
S = 2048
N = B * S

NSPLIT = 2
NP = N // NSPLIT
BH = B // NSPLIT

NC = 2
NS = 16
NW = NC * NS
ROWS_PER_W = NP // NW
CHUNK = 64
NCHUNK = ROWS_PER_W // CHUNK


def _sc_gather_kernel(piece, table_hbm, idx_hbm, out_hbm,
                      idx0, idx1, buf0, buf1, sem0, sem1):
  wid = lax.axis_index("s") * NC + lax.axis_index("c")
  base = pl.multiple_of(wid * ROWS_PER_W, ROWS_PER_W)

  idxs = (idx0, idx1)
  bufs = (buf0, buf1)
  sems = (sem0, sem1)

  def start(ci):
    off = pl.multiple_of(base + ci * CHUNK, CHUNK)
    slot = ci % 2
    grow = piece * NP + off
    b = lax.div(grow, S)
    col = pl.multiple_of(lax.rem(grow, S), CHUNK)
    pltpu.sync_copy(idx_hbm.at[b, pl.ds(col, CHUNK)], idxs[slot])
    return pltpu.async_copy(table_hbm.at[idxs[slot]], bufs[slot], sems[slot])

  cp = start(0)
  for ci in range(NCHUNK):
    nxt = start(ci + 1) if ci + 1 < NCHUNK else None
    cp.wait()
    off = pl.multiple_of(base + ci * CHUNK, CHUNK)
    pltpu.sync_copy(bufs[ci % 2], out_hbm.at[pl.ds(off, CHUNK)])
    cp = nxt


def _make_sc_gather(piece):
  mesh = plsc.VectorSubcoreMesh(core_axis_name="c", subcore_axis_name="s")
  k = functools.partial(
      pl.kernel, mesh=mesh,
      out_type=jax.ShapeDtypeStruct((NP, DIM), jnp.float32),
      scratch_types=[
          pltpu.VMEM((CHUNK,), jnp.int32),
          pltpu.VMEM((CHUNK,), jnp.int32),
          pltpu.VMEM((CHUNK, DIM), jnp.float32),
          pltpu.VMEM((CHUNK, DIM), jnp.float32),
          pltpu.SemaphoreType.DMA,
          pltpu.SemaphoreType.DMA,
      ],
  )(functools.partial(_sc_gather_kernel, piece))
  return k


BS = 1024
SB = S // BS
PB = NP // BS


def _tc_ln_kernel(g_ref, pos_ref, tt_ref, w01_ref, gamma_ref, beta_ref,
                  out_ref):
  row0 = w01_ref[0, :]
  drow = w01_ref[1, :] - row0
  x = g_ref[...] + pos_ref[...]
  x = x + row0[None, :] + tt_ref[...] * drow[None, :]
  mean = jnp.mean(x, axis=-1, keepdims=True)
  xc = x - mean
  var = jnp.mean(xc * xc, axis=-1, keepdims=True)
  y = xc * lax.rsqrt(var + 1e-5)
  out_ref[...] = y * gamma_ref[...] + beta_ref[...]


def _tc_ln_kernel_aliased(g_ref, pos_ref, tt_ref, w01_ref, gamma_ref,
                          beta_ref, prev_ref, out_ref):
  del prev_ref
  _tc_ln_kernel(g_ref, pos_ref, tt_ref, w01_ref, gamma_ref, beta_ref,
                out_ref)


def _make_tc_ln(piece):
  first = piece == 0
  body = _tc_ln_kernel if first else _tc_ln_kernel_aliased
  in_specs = [
      pl.BlockSpec((BS, DIM), lambda s, b: (b * SB + s, 0)),
      pl.BlockSpec((BS, DIM), lambda s, b: (s, 0)),
      pl.BlockSpec((BS, 1),
                   lambda s, b, p=piece: ((p * BH + b) * SB + s, 0)),
      pl.BlockSpec((8, DIM), lambda s, b: (0, 0)),
      pl.BlockSpec((1, DIM), lambda s, b: (0, 0)),
      pl.BlockSpec((1, DIM), lambda s, b: (0, 0)),
  ]
  if not first:
    in_specs.append(pl.BlockSpec((8, DIM), lambda s, b: (0, 0)))
  return pl.pallas_call(
      body,
      grid=(SB, BH),
      in_specs=in_specs,
      out_specs=pl.BlockSpec(
          (BS, DIM), lambda s, b, p=piece: ((p * BH + b) * SB + s, 0)),
      out_shape=jax.ShapeDtypeStruct((N, DIM), jnp.float32),
      input_output_aliases={} if first else {6: 0},
  )


@jax.jit
def _run(input_ids, ttf, W_word, W_tt, gamma2d, beta2d):
  gs = [_make_sc_gather(p)(W_word, input_ids) for p in range(NSPLIT)]
  out = _make_tc_ln(0)(gs[0], W_tt, ttf, W_tt, gamma2d, beta2d)
  for p in range(1, NSPLIT):
    out = _make_tc_ln(p)(gs[p], W_tt, ttf, W_tt, gamma2d, beta2d, out)
  return out


def kernel(input_ids, token_type_ids, W_word, W_tt, gamma, beta):
  ids2d = input_ids.astype(jnp.int32)
  ttf = token_type_ids.reshape(N, 1).astype(jnp.float32)
  out = _run(ids2d, ttf, W_word, W_tt,
             gamma.reshape(1, DIM), beta.reshape(1, DIM))
  return out.reshape(B, S, DIM)


# speedup vs baseline: 1.1919x; 1.0235x over previous
import functools

import jax
import jax.numpy as jnp
from jax import lax
from jax.experimental import pallas as pl
from jax.experimental.pallas import tpu as pltpu
from jax.experimental.pallas import tpu_sc as plsc

VOCAB = 100000
MAXLEN = 2048
DIM = 768
B = 4
S = 2048
N = B * S

NSPLIT = 2
NP = N // NSPLIT
BH = B // NSPLIT

NC = 2
NS = 16
NW = NC * NS
ROWS_PER_W = NP // NW
CHUNK = 64
NCHUNK = ROWS_PER_W // CHUNK


def _sc_gather_kernel(piece, table_hbm, idx_hbm, out_hbm,
                      idx0, idx1, buf0, buf1, sem0, sem1):
  wid = lax.axis_index("s") * NC + lax.axis_index("c")
  base = pl.multiple_of(wid * ROWS_PER_W, ROWS_PER_W)

  idxs = (idx0, idx1)
  bufs = (buf0, buf1)
  sems = (sem0, sem1)

  def start(ci):
    off = pl.multiple_of(base + ci * CHUNK, CHUNK)
    slot = ci % 2
    grow = piece * NP + off
    b = lax.div(grow, S)
    col = pl.multiple_of(lax.rem(grow, S), CHUNK)
    pltpu.sync_copy(idx_hbm.at[b, pl.ds(col, CHUNK)], idxs[slot])
    return pltpu.async_copy(table_hbm.at[idxs[slot]], bufs[slot], sems[slot])

  cp = start(0)
  for ci in range(NCHUNK):
    nxt = start(ci + 1) if ci + 1 < NCHUNK else None
    cp.wait()
    off = pl.multiple_of(base + ci * CHUNK, CHUNK)
    pltpu.sync_copy(bufs[ci % 2], out_hbm.at[pl.ds(off, CHUNK)])
    cp = nxt


def _make_sc_gather(piece):
  mesh = plsc.VectorSubcoreMesh(core_axis_name="c", subcore_axis_name="s")
  k = functools.partial(
      pl.kernel, mesh=mesh,
      out_type=jax.ShapeDtypeStruct((NP, DIM), jnp.float32),
      scratch_types=[
          pltpu.VMEM((CHUNK,), jnp.int32),
          pltpu.VMEM((CHUNK,), jnp.int32),
          pltpu.VMEM((CHUNK, DIM), jnp.float32),
          pltpu.VMEM((CHUNK, DIM), jnp.float32),
          pltpu.SemaphoreType.DMA,
          pltpu.SemaphoreType.DMA,
      ],
  )(functools.partial(_sc_gather_kernel, piece))
  return k


BS = 2048
SB = S // BS
PB = NP // BS


def _tc_ln_kernel(g_ref, pos_ref, tt_ref, w01_ref, gamma_ref, beta_ref,
                  out_ref):
  row0 = w01_ref[0, :]
  drow = w01_ref[1, :] - row0
  x = g_ref[...] + pos_ref[...]
  x = x + row0[None, :] + tt_ref[...] * drow[None, :]
  mean = jnp.mean(x, axis=-1, keepdims=True)
  xc = x - mean
  var = jnp.mean(xc * xc, axis=-1, keepdims=True)
  y = xc * lax.rsqrt(var + 1e-5)
  out_ref[...] = y * gamma_ref[...] + beta_ref[...]


def _tc_ln_kernel_aliased(g_ref, pos_ref, tt_ref, w01_ref, gamma_ref,
                          beta_ref, prev_ref, out_ref):
  del prev_ref
  _tc_ln_kernel(g_ref, pos_ref, tt_ref, w01_ref, gamma_ref, beta_ref,
                out_ref)


def _make_tc_ln(piece):
  first = piece == 0
  body = _tc_ln_kernel if first else _tc_ln_kernel_aliased
  in_specs = [
      pl.BlockSpec((BS, DIM), lambda s, b: (b * SB + s, 0)),
      pl.BlockSpec((BS, DIM), lambda s, b: (s, 0)),
      pl.BlockSpec((BS, 1),
                   lambda s, b, p=piece: ((p * BH + b) * SB + s, 0)),
      pl.BlockSpec((8, DIM), lambda s, b: (0, 0)),
      pl.BlockSpec((1, DIM), lambda s, b: (0, 0)),
      pl.BlockSpec((1, DIM), lambda s, b: (0, 0)),
  ]
  if not first:
    in_specs.append(pl.BlockSpec((8, DIM), lambda s, b: (0, 0)))
  return pl.pallas_call(
      body,
      grid=(SB, BH),
      in_specs=in_specs,
      out_specs=pl.BlockSpec(
          (BS, DIM), lambda s, b, p=piece: ((p * BH + b) * SB + s, 0)),
      out_shape=jax.ShapeDtypeStruct((N, DIM), jnp.float32),
      input_output_aliases={} if first else {6: 0},
  )


@jax.jit
def _run(input_ids, ttf, W_word, W_tt, gamma2d, beta2d):
  gs = [_make_sc_gather(p)(W_word, input_ids) for p in range(NSPLIT)]
  out = _make_tc_ln(0)(gs[0], W_tt, ttf, W_tt, gamma2d, beta2d)
  for p in range(1, NSPLIT):
    out = _make_tc_ln(p)(gs[p], W_tt, ttf, W_tt, gamma2d, beta2d, out)
  return out


def kernel(input_ids, token_type_ids, W_word, W_tt, gamma, beta):
  ids2d = input_ids.astype(jnp.int32)
  ttf = token_type_ids.reshape(N, 1).astype(jnp.float32)
  out = _run(ids2d, ttf, W_word, W_tt,
             gamma.reshape(1, DIM), beta.reshape(1, DIM))
  return out.reshape(B, S, DIM)


# speedup vs baseline: 1.1954x; 1.0029x over previous
import functools

import jax
import jax.numpy as jnp
from jax import lax
from jax.experimental import pallas as pl
from jax.experimental.pallas import tpu as pltpu
from jax.experimental.pallas import tpu_sc as plsc

VOCAB = 100000
MAXLEN = 2048
DIM = 768
B = 4
S = 2048
N = B * S

NSPLIT = 2
NP = N // NSPLIT
BH = B // NSPLIT

NC = 2
NS = 16
NW = NC * NS
ROWS_PER_W = NP // NW
CHUNK = 64
NCHUNK = ROWS_PER_W // CHUNK


def _sc_gather_kernel(piece, table_hbm, idx_hbm, out_hbm,
                      idx0, idx1, buf0, buf1, sem0, sem1, osem0, osem1):
  wid = lax.axis_index("s") * NC + lax.axis_index("c")
  base = pl.multiple_of(wid * ROWS_PER_W, ROWS_PER_W)

  idxs = (idx0, idx1)
  bufs = (buf0, buf1)
  sems = (sem0, sem1)
  osems = (osem0, osem1)

  def start(ci):
    off = pl.multiple_of(base + ci * CHUNK, CHUNK)
    slot = ci % 2
    grow = piece * NP + off
    b = lax.div(grow, S)
    col = pl.multiple_of(lax.rem(grow, S), CHUNK)
    pltpu.sync_copy(idx_hbm.at[b, pl.ds(col, CHUNK)], idxs[slot])
    return pltpu.async_copy(table_hbm.at[idxs[slot]], bufs[slot], sems[slot])

  gathers = [start(ci) for ci in range(NCHUNK)]
  outs = []
  for ci in range(NCHUNK):
    gathers[ci].wait()
    off = pl.multiple_of(base + ci * CHUNK, CHUNK)
    outs.append(pltpu.async_copy(bufs[ci % 2],
                                 out_hbm.at[pl.ds(off, CHUNK)],
                                 osems[ci % 2]))
  for cp in outs:
    cp.wait()


def _make_sc_gather(piece):
  mesh = plsc.VectorSubcoreMesh(core_axis_name="c", subcore_axis_name="s")
  k = functools.partial(
      pl.kernel, mesh=mesh,
      out_type=jax.ShapeDtypeStruct((NP, DIM), jnp.float32),
      scratch_types=[
          pltpu.VMEM((CHUNK,), jnp.int32),
          pltpu.VMEM((CHUNK,), jnp.int32),
          pltpu.VMEM((CHUNK, DIM), jnp.float32),
          pltpu.VMEM((CHUNK, DIM), jnp.float32),
          pltpu.SemaphoreType.DMA,
          pltpu.SemaphoreType.DMA,
          pltpu.SemaphoreType.DMA,
          pltpu.SemaphoreType.DMA,
      ],
  )(functools.partial(_sc_gather_kernel, piece))
  return k


BS = 2048
SB = S // BS
PB = NP // BS


def _tc_ln_kernel(g_ref, pos_ref, tt_ref, w01_ref, gamma_ref, beta_ref,
                  out_ref):
  row0 = w01_ref[0, :]
  drow = w01_ref[1, :] - row0
  x = g_ref[...] + pos_ref[...]
  x = x + row0[None, :] + tt_ref[...] * drow[None, :]
  mean = jnp.mean(x, axis=-1, keepdims=True)
  xc = x - mean
  var = jnp.mean(xc * xc, axis=-1, keepdims=True)
  y = xc * lax.rsqrt(var + 1e-5)
  out_ref[...] = y * gamma_ref[...] + beta_ref[...]


def _tc_ln_kernel_aliased(g_ref, pos_ref, tt_ref, w01_ref, gamma_ref,
                          beta_ref, prev_ref, out_ref):
  del prev_ref
  _tc_ln_kernel(g_ref, pos_ref, tt_ref, w01_ref, gamma_ref, beta_ref,
                out_ref)


def _make_tc_ln(piece):
  first = piece == 0
  body = _tc_ln_kernel if first else _tc_ln_kernel_aliased
  in_specs = [
      pl.BlockSpec((BS, DIM), lambda s, b: (b * SB + s, 0)),
      pl.BlockSpec((BS, DIM), lambda s, b: (s, 0)),
      pl.BlockSpec((BS, 1),
                   lambda s, b, p=piece: ((p * BH + b) * SB + s, 0)),
      pl.BlockSpec((8, DIM), lambda s, b: (0, 0)),
      pl.BlockSpec((1, DIM), lambda s, b: (0, 0)),
      pl.BlockSpec((1, DIM), lambda s, b: (0, 0)),
  ]
  if not first:
    in_specs.append(pl.BlockSpec((8, DIM), lambda s, b: (0, 0)))
  return pl.pallas_call(
      body,
      grid=(SB, BH),
      in_specs=in_specs,
      out_specs=pl.BlockSpec(
          (BS, DIM), lambda s, b, p=piece: ((p * BH + b) * SB + s, 0)),
      out_shape=jax.ShapeDtypeStruct((N, DIM), jnp.float32),
      input_output_aliases={} if first else {6: 0},
  )


@jax.jit
def _run(input_ids, ttf, W_word, W_tt, gamma2d, beta2d):
  gs = [_make_sc_gather(p)(W_word, input_ids) for p in range(NSPLIT)]
  out = _make_tc_ln(0)(gs[0], W_tt, ttf, W_tt, gamma2d, beta2d)
  for p in range(1, NSPLIT):
    out = _make_tc_ln(p)(gs[p], W_tt, ttf, W_tt, gamma2d, beta2d, out)
  return out


def kernel(input_ids, token_type_ids, W_word, W_tt, gamma, beta):
  ids2d = input_ids.astype(jnp.int32)
  ttf = token_type_ids.reshape(N, 1).astype(jnp.float32)
  out = _run(ids2d, ttf, W_word, W_tt,
             gamma.reshape(1, DIM), beta.reshape(1, DIM))
  return out.reshape(B, S, DIM)
